# baseline (device time: 172583 ns/iter reference)
import os

import jax
import jax.numpy as jnp
from jax import lax
from jax.experimental import pallas as pl
from jax.experimental.pallas import tpu as pltpu

_SKIP_COMM = os.environ.get("KERNEL_SKIP_COMM") == "1"

N_DEV = 4
M_BLK = 1024
K = 4096
N = 8192
TN = 1024
NT = N // TN
N_TILES = N_DEV * NT
W_DEPTH = 4


def kernel(x, w_mat):
    assert x.shape == (K, M_BLK), (x.shape,)
    assert w_mat.shape == (K, N)
    x = x.astype(jnp.bfloat16)

    def body(x_ref, w_ref, out_ref, g_ref, xloc_ref, wbuf_ref, amax_ref,
             x_send, x_recv, a_send, a_recv, w_sems, xl_sem):
        me = lax.axis_index("i")

        sends = []
        if not _SKIP_COMM:
            barrier_sem = pltpu.get_barrier_semaphore()
            for d in range(1, N_DEV):
                pl.semaphore_signal(
                    barrier_sem, inc=1,
                    device_id=((me + d) % N_DEV,),
                    device_id_type=pl.DeviceIdType.MESH,
                )
            pl.semaphore_wait(barrier_sem, N_DEV - 1)

            for d in range(1, N_DEV):
                dst = (me + d) % N_DEV
                rdma = pltpu.make_async_remote_copy(
                    src_ref=x_ref.at[pl.ds(dst * M_BLK, M_BLK), :],
                    dst_ref=g_ref.at[3 - d],
                    send_sem=x_send.at[d - 1],
                    recv_sem=x_recv.at[3 - d],
                    device_id=(dst,),
                    device_id_type=pl.DeviceIdType.MESH,
                )
                rdma.start()
                sends.append(rdma)

        xl_cp = pltpu.make_async_copy(
            x_ref.at[pl.ds(me * M_BLK, M_BLK), :], xloc_ref, xl_sem)
        xl_cp.start()

        offsets = [0, 1, 3, 2]
        srcs = [(me + doff) % N_DEV for doff in offsets]

        def start_w(t):
            j, nt = divmod(t, NT)
            slot = t % W_DEPTH
            cp = pltpu.make_async_copy(
                w_ref.at[pl.ds(srcs[j] * M_BLK, M_BLK),
                         pl.ds(nt * TN, TN)],
                wbuf_ref.at[slot],
                w_sems.at[slot],
            )
            cp.start()
            return cp

        amax = jnp.float32(0.0)
        xblk = None
        cps = [start_w(t) for t in range(W_DEPTH - 1)] + [None]
        for t in range(N_TILES):
            j, nt = divmod(t, NT)
            slot = t % W_DEPTH
            if t + W_DEPTH - 1 < N_TILES:
                cps[(t + W_DEPTH - 1) % W_DEPTH] = start_w(t + W_DEPTH - 1)
            if nt == 0:
                doff = offsets[j]
                if doff == 0:
                    xl_cp.wait()
                    xblk = xloc_ref[...]
                elif _SKIP_COMM:
                    xblk = xloc_ref[...]
                else:
                    recv = pltpu.make_async_remote_copy(
                        src_ref=g_ref.at[doff - 1],
                        dst_ref=g_ref.at[doff - 1],
                        send_sem=x_send.at[0],
                        recv_sem=x_recv.at[doff - 1],
                        device_id=(srcs[j],),
                        device_id_type=pl.DeviceIdType.MESH,
                    )
                    recv.wait_recv()
                    xblk = g_ref[doff - 1]
            cps[slot].wait()
            part = jnp.dot(
                xblk, wbuf_ref[slot].astype(jnp.bfloat16),
                preferred_element_type=jnp.float32,
            )
            nsl = pl.ds(nt * TN, TN)
            if j == 0:
                out_ref[:, nsl] = part
            else:
                tile = out_ref[:, nsl] + part
                out_ref[:, nsl] = tile
                if j == N_DEV - 1:
                    amax = jnp.maximum(amax, jnp.max(tile))

        amax_ref[0] = jnp.full((8, 128), amax, dtype=jnp.float32)
        a_sends = []
        for d in range(1, N_DEV) if not _SKIP_COMM else []:
            dst = (me + d) % N_DEV
            rdma = pltpu.make_async_remote_copy(
                src_ref=amax_ref.at[0],
                dst_ref=amax_ref.at[N_DEV - d],
                send_sem=a_send.at[d - 1],
                recv_sem=a_recv.at[N_DEV - d],
                device_id=(dst,),
                device_id_type=pl.DeviceIdType.MESH,
            )
            rdma.start()
            a_sends.append(rdma)
        for d in range(1, N_DEV) if not _SKIP_COMM else []:
            recv = pltpu.make_async_remote_copy(
                src_ref=amax_ref.at[d],
                dst_ref=amax_ref.at[d],
                send_sem=a_send.at[0],
                recv_sem=a_recv.at[d],
                device_id=((me + d) % N_DEV,),
                device_id_type=pl.DeviceIdType.MESH,
            )
            recv.wait_recv()
            amax = jnp.maximum(amax, amax_ref[d, 0, 0])

        scale = jnp.maximum(amax, jnp.float32(1e-30)) / 127.0
        for nt in range(NT):
            nsl = pl.ds(nt * TN, TN)
            y = jnp.maximum(out_ref[:, nsl], 0.0)
            q = jnp.clip(jnp.round(y / scale), -127.0, 127.0)
            out_ref[:, nsl] = q * scale

        for rdma in sends + a_sends:
            rdma.wait_send()

    return pl.pallas_call(
        body,
        out_shape=jax.ShapeDtypeStruct((M_BLK, N), jnp.float32),
        in_specs=[
            pl.BlockSpec(memory_space=pl.ANY),
            pl.BlockSpec(memory_space=pl.ANY),
        ],
        out_specs=pl.BlockSpec(memory_space=pltpu.MemorySpace.VMEM),
        scratch_shapes=[
            pltpu.VMEM((N_DEV - 1, M_BLK, M_BLK), jnp.bfloat16),
            pltpu.VMEM((M_BLK, M_BLK), jnp.bfloat16),
            pltpu.VMEM((W_DEPTH, M_BLK, TN), jnp.float32),
            pltpu.VMEM((N_DEV, 8, 128), jnp.float32),
            pltpu.SemaphoreType.DMA((N_DEV - 1,)),
            pltpu.SemaphoreType.DMA((N_DEV - 1,)),
            pltpu.SemaphoreType.DMA((N_DEV - 1,)),
            pltpu.SemaphoreType.DMA((N_DEV,)),
            pltpu.SemaphoreType.DMA((W_DEPTH,)),
            pltpu.SemaphoreType.DMA(()),
        ],
        compiler_params=pltpu.CompilerParams(
            collective_id=None if _SKIP_COMM else 0,
            vmem_limit_bytes=110 * 1024 * 1024,
        ),
    )(x, w_mat)


# device time: 153601 ns/iter; 1.1236x vs baseline; 1.1236x over previous
import os

import jax
import jax.numpy as jnp
from jax import lax
from jax.experimental import pallas as pl
from jax.experimental.pallas import tpu as pltpu

_SKIP_COMM = os.environ.get("KERNEL_SKIP_COMM") == "1"

N_DEV = 4
M_BLK = 1024
K = 4096
N = 8192
TN = 1024
NT = N // TN
N_TILES = N_DEV * NT
W_DEPTH = 3


def kernel(x, w_mat):
    assert x.shape == (K, M_BLK), (x.shape,)
    assert w_mat.shape == (K, N)
    x = x.astype(jnp.bfloat16)

    def body(x_ref, w_ref, out_ref, g_ref, xloc_ref, wbuf_ref, wbf_ref,
             amax_ref, x_send, x_recv, a_send, a_recv, w_sems, xl_sem):
        me = lax.axis_index("i")

        sends = []
        if not _SKIP_COMM:
            barrier_sem = pltpu.get_barrier_semaphore()
            for d in range(1, N_DEV):
                pl.semaphore_signal(
                    barrier_sem, inc=1,
                    device_id=((me + d) % N_DEV,),
                    device_id_type=pl.DeviceIdType.MESH,
                )
            pl.semaphore_wait(barrier_sem, N_DEV - 1)

            for d in (3, 1, 2):
                dst = (me + d) % N_DEV
                rdma = pltpu.make_async_remote_copy(
                    src_ref=x_ref.at[pl.ds(dst * M_BLK, M_BLK), :],
                    dst_ref=g_ref.at[3 - d],
                    send_sem=x_send.at[d - 1],
                    recv_sem=x_recv.at[3 - d],
                    device_id=(dst,),
                    device_id_type=pl.DeviceIdType.MESH,
                )
                rdma.start()
                sends.append(rdma)

        xl_cp = pltpu.make_async_copy(
            x_ref.at[pl.ds(me * M_BLK, M_BLK), :], xloc_ref, xl_sem)
        xl_cp.start()

        offsets = [0, 1, 3, 2]
        srcs = [(me + doff) % N_DEV for doff in offsets]

        def start_w(t):
            j, nt = divmod(t, NT)
            slot = t % W_DEPTH
            cp = pltpu.make_async_copy(
                w_ref.at[pl.ds(srcs[j] * M_BLK, M_BLK),
                         pl.ds(nt * TN, TN)],
                wbuf_ref.at[slot],
                w_sems.at[slot],
            )
            cp.start()
            return cp

        amax = jnp.float32(0.0)
        xblk = None
        cps = [start_w(t) for t in range(W_DEPTH - 1)] + [None]
        pending = None
        for t in range(N_TILES):
            j, nt = divmod(t, NT)
            slot = t % W_DEPTH
            if t + W_DEPTH - 1 < N_TILES:
                cps[(t + W_DEPTH - 1) % W_DEPTH] = start_w(t + W_DEPTH - 1)
            if nt == 0:
                doff = offsets[j]
                if doff == 0:
                    xl_cp.wait()
                    xblk = xloc_ref[...]
                elif _SKIP_COMM:
                    xblk = xloc_ref[...]
                else:
                    recv = pltpu.make_async_remote_copy(
                        src_ref=g_ref.at[doff - 1],
                        dst_ref=g_ref.at[doff - 1],
                        send_sem=x_send.at[0],
                        recv_sem=x_recv.at[doff - 1],
                        device_id=(srcs[j],),
                        device_id_type=pl.DeviceIdType.MESH,
                    )
                    recv.wait_recv()
                    xblk = g_ref[doff - 1]
            cps[slot].wait()
            wbf_ref[t % 2] = wbuf_ref[slot].astype(jnp.bfloat16)
            part = jnp.dot(
                xblk, wbf_ref[t % 2],
                preferred_element_type=jnp.float32,
            )
            if pending is not None:
                pj, pnt, ppart = pending
                nsl = pl.ds(pnt * TN, TN)
                if pj == 0:
                    out_ref[:, nsl] = ppart
                else:
                    tile = out_ref[:, nsl] + ppart
                    out_ref[:, nsl] = tile
                    if pj == N_DEV - 1:
                        amax = jnp.maximum(amax, jnp.max(tile))
            pending = (j, nt, part)
        pj, pnt, ppart = pending
        nsl = pl.ds(pnt * TN, TN)
        tile = out_ref[:, nsl] + ppart
        out_ref[:, nsl] = tile
        amax = jnp.maximum(amax, jnp.max(tile))

        amax_ref[0] = jnp.full((8, 128), amax, dtype=jnp.float32)
        a_sends = []
        for d in range(1, N_DEV) if not _SKIP_COMM else []:
            dst = (me + d) % N_DEV
            rdma = pltpu.make_async_remote_copy(
                src_ref=amax_ref.at[0],
                dst_ref=amax_ref.at[N_DEV - d],
                send_sem=a_send.at[d - 1],
                recv_sem=a_recv.at[N_DEV - d],
                device_id=(dst,),
                device_id_type=pl.DeviceIdType.MESH,
            )
            rdma.start()
            a_sends.append(rdma)
        for d in range(1, N_DEV) if not _SKIP_COMM else []:
            recv = pltpu.make_async_remote_copy(
                src_ref=amax_ref.at[d],
                dst_ref=amax_ref.at[d],
                send_sem=a_send.at[0],
                recv_sem=a_recv.at[d],
                device_id=((me + d) % N_DEV,),
                device_id_type=pl.DeviceIdType.MESH,
            )
            recv.wait_recv()
            amax = jnp.maximum(amax, amax_ref[d, 0, 0])

        scale = jnp.maximum(amax, jnp.float32(1e-30)) / 127.0
        for nt in range(NT):
            nsl = pl.ds(nt * TN, TN)
            y = jnp.maximum(out_ref[:, nsl], 0.0)
            q = jnp.clip(jnp.round(y / scale), -127.0, 127.0)
            out_ref[:, nsl] = q * scale

        for rdma in sends + a_sends:
            rdma.wait_send()

    return pl.pallas_call(
        body,
        out_shape=jax.ShapeDtypeStruct((M_BLK, N), jnp.float32),
        in_specs=[
            pl.BlockSpec(memory_space=pl.ANY),
            pl.BlockSpec(memory_space=pl.ANY),
        ],
        out_specs=pl.BlockSpec(memory_space=pltpu.MemorySpace.VMEM),
        scratch_shapes=[
            pltpu.VMEM((N_DEV - 1, M_BLK, M_BLK), jnp.bfloat16),
            pltpu.VMEM((M_BLK, M_BLK), jnp.bfloat16),
            pltpu.VMEM((W_DEPTH, M_BLK, TN), jnp.float32),
            pltpu.VMEM((2, M_BLK, TN), jnp.bfloat16),
            pltpu.VMEM((N_DEV, 8, 128), jnp.float32),
            pltpu.SemaphoreType.DMA((N_DEV - 1,)),
            pltpu.SemaphoreType.DMA((N_DEV - 1,)),
            pltpu.SemaphoreType.DMA((N_DEV - 1,)),
            pltpu.SemaphoreType.DMA((N_DEV,)),
            pltpu.SemaphoreType.DMA((W_DEPTH,)),
            pltpu.SemaphoreType.DMA(()),
        ],
        compiler_params=pltpu.CompilerParams(
            collective_id=None if _SKIP_COMM else 0,
            vmem_limit_bytes=110 * 1024 * 1024,
        ),
    )(x, w_mat)


# device time: 149591 ns/iter; 1.1537x vs baseline; 1.0268x over previous
import os

import jax
import jax.numpy as jnp
from jax import lax
from jax.experimental import pallas as pl
from jax.experimental.pallas import tpu as pltpu

_SKIP_COMM = os.environ.get("KERNEL_SKIP_COMM") == "1"

N_DEV = 4
M_BLK = 1024
K = 4096
N = 8192
TN = 1024
NT = N // TN
N_TILES = N_DEV * NT
W_DEPTH = 3


def kernel(x, w_mat):
    assert x.shape == (K, M_BLK), (x.shape,)
    assert w_mat.shape == (K, N)
    x = x.astype(jnp.bfloat16)

    def body(x_ref, w_ref, out_ref, g_ref, xloc_ref, wbuf_ref,
             wbf_ref, amax_ref, x_send, x_recv, a_send, a_recv, w_sems,
             xl_sem):
        me = lax.axis_index("i")

        sends = []
        if not _SKIP_COMM:
            barrier_sem = pltpu.get_barrier_semaphore()
            for d in range(1, N_DEV):
                pl.semaphore_signal(
                    barrier_sem, inc=1,
                    device_id=((me + d) % N_DEV,),
                    device_id_type=pl.DeviceIdType.MESH,
                )
            pl.semaphore_wait(barrier_sem, N_DEV - 1)

            diag_rdma = None
            for d in (3, 1, 2):
                dst = (me + d) % N_DEV
                rdma = pltpu.make_async_remote_copy(
                    src_ref=x_ref.at[pl.ds(dst * M_BLK, M_BLK), :],
                    dst_ref=g_ref.at[3 - d],
                    send_sem=x_send.at[d - 1],
                    recv_sem=x_recv.at[3 - d],
                    device_id=(dst,),
                    device_id_type=pl.DeviceIdType.MESH,
                )
                if d == 2:
                    diag_rdma = rdma
                else:
                    rdma.start()
                sends.append(rdma)

        xl_cp = pltpu.make_async_copy(
            x_ref.at[pl.ds(me * M_BLK, M_BLK), :], xloc_ref, xl_sem)
        xl_cp.start()

        offsets = [0, 1, 3, 2]
        srcs = [(me + doff) % N_DEV for doff in offsets]

        def start_w(t):
            j, nt = divmod(t, NT)
            slot = t % W_DEPTH
            cp = pltpu.make_async_copy(
                w_ref.at[pl.ds(srcs[j] * M_BLK, M_BLK),
                         pl.ds(nt * TN, TN)],
                wbuf_ref.at[slot],
                w_sems.at[slot],
            )
            cp.start()
            return cp

        amax = jnp.float32(0.0)
        xblk = None
        cps = [start_w(t) for t in range(W_DEPTH - 1)] + [None]
        pending = None
        for t in range(N_TILES):
            j, nt = divmod(t, NT)
            slot = t % W_DEPTH
            if t + W_DEPTH - 1 < N_TILES:
                cps[(t + W_DEPTH - 1) % W_DEPTH] = start_w(t + W_DEPTH - 1)
            if t == NT and not _SKIP_COMM:
                diag_rdma.start()
            if nt == 0:
                doff = offsets[j]
                if doff == 0:
                    xl_cp.wait()
                    xblk = xloc_ref[...]
                elif _SKIP_COMM:
                    xblk = xloc_ref[...]
                else:
                    recv = pltpu.make_async_remote_copy(
                        src_ref=g_ref.at[doff - 1],
                        dst_ref=g_ref.at[doff - 1],
                        send_sem=x_send.at[0],
                        recv_sem=x_recv.at[doff - 1],
                        device_id=(srcs[j],),
                        device_id_type=pl.DeviceIdType.MESH,
                    )
                    recv.wait_recv()
                    xblk = g_ref[doff - 1]
            cps[slot].wait()
            wbf_ref[t % 2] = wbuf_ref[slot].astype(jnp.bfloat16)
            part = jnp.dot(
                xblk, wbf_ref[t % 2],
                preferred_element_type=jnp.float32,
            )
            if pending is not None:
                pj, pnt, ppart = pending
                nsl = pl.ds(pnt * TN, TN)
                if pj == 0:
                    out_ref[:, nsl] = ppart
                else:
                    tile = out_ref[:, nsl] + ppart
                    out_ref[:, nsl] = tile
                    if pj == N_DEV - 1:
                        amax = jnp.maximum(amax, jnp.max(tile))
            pending = (j, nt, part)
        pj, pnt, ppart = pending
        nsl = pl.ds(pnt * TN, TN)
        tile = out_ref[:, nsl] + ppart
        out_ref[:, nsl] = tile
        amax = jnp.maximum(amax, jnp.max(tile))

        amax_ref[0] = jnp.full((8, 128), amax, dtype=jnp.float32)
        a_sends = []
        for d in range(1, N_DEV) if not _SKIP_COMM else []:
            dst = (me + d) % N_DEV
            rdma = pltpu.make_async_remote_copy(
                src_ref=amax_ref.at[0],
                dst_ref=amax_ref.at[N_DEV - d],
                send_sem=a_send.at[d - 1],
                recv_sem=a_recv.at[N_DEV - d],
                device_id=(dst,),
                device_id_type=pl.DeviceIdType.MESH,
            )
            rdma.start()
            a_sends.append(rdma)
        for d in range(1, N_DEV) if not _SKIP_COMM else []:
            recv = pltpu.make_async_remote_copy(
                src_ref=amax_ref.at[d],
                dst_ref=amax_ref.at[d],
                send_sem=a_send.at[0],
                recv_sem=a_recv.at[d],
                device_id=((me + d) % N_DEV,),
                device_id_type=pl.DeviceIdType.MESH,
            )
            recv.wait_recv()
            amax = jnp.maximum(amax, amax_ref[d, 0, 0])

        scale = jnp.maximum(amax, jnp.float32(1e-30)) / 127.0
        for nt in range(NT):
            nsl = pl.ds(nt * TN, TN)
            y = jnp.maximum(out_ref[:, nsl], 0.0)
            q = jnp.clip(jnp.round(y / scale), -127.0, 127.0)
            out_ref[:, nsl] = q * scale

        for rdma in sends + a_sends:
            rdma.wait_send()

    return pl.pallas_call(
        body,
        out_shape=jax.ShapeDtypeStruct((M_BLK, N), jnp.float32),
        in_specs=[
            pl.BlockSpec(memory_space=pl.ANY),
            pl.BlockSpec(memory_space=pl.ANY),
        ],
        out_specs=pl.BlockSpec(memory_space=pltpu.MemorySpace.VMEM),
        scratch_shapes=[
            pltpu.VMEM((N_DEV - 1, M_BLK, M_BLK), jnp.bfloat16),
            pltpu.VMEM((M_BLK, M_BLK), jnp.bfloat16),
            pltpu.VMEM((W_DEPTH, M_BLK, TN), jnp.float32),
            pltpu.VMEM((2, M_BLK, TN), jnp.bfloat16),
            pltpu.VMEM((N_DEV, 8, 128), jnp.float32),
            pltpu.SemaphoreType.DMA((N_DEV - 1,)),
            pltpu.SemaphoreType.DMA((N_DEV - 1,)),
            pltpu.SemaphoreType.DMA((N_DEV - 1,)),
            pltpu.SemaphoreType.DMA((N_DEV,)),
            pltpu.SemaphoreType.DMA((W_DEPTH,)),
            pltpu.SemaphoreType.DMA(()),
        ],
        compiler_params=pltpu.CompilerParams(
            collective_id=None if _SKIP_COMM else 0,
            vmem_limit_bytes=110 * 1024 * 1024,
        ),
    )(x, w_mat)
